# baseline (speedup 1.0000x reference)
import jax
import jax.numpy as jnp
from jax.experimental import pallas as pl
from jax.experimental.pallas import tpu as pltpu

_RB = 512


def _add_kernel(x_ref, row_ref, o_ref, *, n_row_blocks):
    off = (pl.program_id(0) % n_row_blocks) * _RB
    o_ref[:, :] = x_ref[:, :] + row_ref[pl.ds(off, _RB), :]


def kernel(x, row_embed):
    B, L, D = x.shape
    table = row_embed[:L]
    x2 = x.reshape(B * L, D)
    from functools import partial

    out = pl.pallas_call(
        partial(_add_kernel, n_row_blocks=L // _RB),
        grid=(B * L // _RB,),
        in_specs=[
            pl.BlockSpec((_RB, D), lambda i: (i, 0)),
            pl.BlockSpec((L, D), lambda i: (0, 0)),
        ],
        out_specs=pl.BlockSpec((_RB, D), lambda i: (i, 0)),
        out_shape=jax.ShapeDtypeStruct((B * L, D), x.dtype),
        compiler_params=pltpu.CompilerParams(
            dimension_semantics=("arbitrary",),
        ),
    )(x2, table)
    return out.reshape(B, L, D)
